# baseline (speedup 1.0000x reference)
import functools

import jax
import jax.numpy as jnp
from jax.experimental import pallas as pl
from jax.experimental.pallas import tpu as pltpu


B, N, F = 256, 256, 53
NF = 36
BB = 64
GRID = B // BB

f32 = jnp.float32
bf16 = jnp.bfloat16


def _dot(a, b, dims=(((1,), (0,)), ((), ()))):
    return jax.lax.dot_general(a, b, dims, preferred_element_type=f32)


def _fused_kernel(x_ref, phys_ref, wrt_ref, bmat_ref, wc_ref, bc_ref, w1_ref,
                  b1_ref, w5_ref, b5_ref, w6_ref, b6_ref, w7_ref, b7_ref,
                  out_ref, pool_ref, a_ref):
    step = pl.program_id(0)

    feats = x_ref[:, :, :NF].reshape(BB * N, NF).astype(bf16)
    fwt = jax.lax.dot_general(wrt_ref[...].astype(bf16), feats,
                              (((1,), (1,)), ((), ())),
                              preferred_element_type=f32)
    bmat = bmat_ref[...]
    ones_row = jnp.ones((1, N), dtype=bf16)

    LAG = 3
    pooled = []
    for i in range(BB + LAG):
        if i < BB:
            f = feats[i * N:(i + 1) * N, :]
            g = jax.lax.dot_general(f, f, (((1,), (1,)), ((), ())),
                                    preferred_element_type=f32)
            a_ref[i, :, :] = jnp.maximum(g, 0.0).astype(bf16)
        if i >= LAG:
            j = i - LAG
            a = a_ref[j, :, :]
            lhs = jnp.concatenate(
                [ones_row,
                 (fwt[:, j * N:(j + 1) * N] + bmat).astype(bf16)], axis=0)
            nt_full = _dot(lhs, a)
            deg = nt_full[0:1, :] + 1e-6
            r = (1.0 / deg).astype(bf16)
            nt = jnp.maximum(nt_full[1:21, :] + 1e-6 * bmat, 0.0)
            pooled.append(jax.lax.dot_general(r, nt.astype(bf16),
                                              (((1,), (1,)), ((), ())),
                                              preferred_element_type=f32))
    pool_ref[pl.ds(step * BB, BB), :] = jnp.concatenate(pooled, axis=0)

    @pl.when(step == GRID - 1)
    def _head():
        p = pool_ref[...].astype(bf16)
        c = jnp.maximum(_dot(p, wc_ref[...].astype(bf16)) + bc_ref[...], 0.0)
        x1 = jnp.maximum(_dot(c.astype(bf16), w1_ref[...].astype(bf16))
                         + b1_ref[...], 0.0)
        x5 = jnp.maximum(_dot(x1.astype(bf16), w5_ref[...].astype(bf16))
                         + b5_ref[...], 0.0)
        mv = _dot(x5.astype(bf16), w6_ref[...].astype(bf16)) + b6_ref[...]
        phys = phys_ref[...]
        col0 = (mv * w7_ref[0, 0]
                + jax.lax.dot_general(phys, w7_ref[1:, :],
                                      (((1,), (0,)), ((), ())),
                                      preferred_element_type=f32)
                + b7_ref[...])
        out_ref[...] = jnp.concatenate([col0, phys], axis=1)


@functools.partial(jax.jit, static_argnames=())
def kernel(inputs, W_rule, b_rule, W_conv, b_conv, W1, b1, W5, b5, W6, b6,
           W7, b7):
    full = lambda shape: pl.BlockSpec(shape, lambda i: (0,) * len(shape))
    phys_all = inputs[:, 0, NF + 2:F]
    b_mat = jnp.broadcast_to(b_rule.reshape(20, 1), (20, N))
    out = pl.pallas_call(
        _fused_kernel,
        grid=(GRID,),
        in_specs=[
            pl.BlockSpec((BB, N, F), lambda i: (i, 0, 0)),
            full((B, 15)),
            full((20, 36)),
            full((20, N)),
            full((20, 1024)),
            full((1, 1024)),
            full((1024, 32)),
            full((1, 32)),
            full((32, 16)),
            full((1, 16)),
            full((16, 1)),
            full((1, 1)),
            full((16, 1)),
            full((1, 1)),
        ],
        out_specs=pl.BlockSpec((B, 16), lambda i: (0, 0)),
        out_shape=jax.ShapeDtypeStruct((B, 16), f32),
        scratch_shapes=[pltpu.VMEM((B, 20), f32),
                        pltpu.VMEM((BB, N, N), bf16)],
    )(inputs, phys_all, W_rule.T, b_mat, W_conv,
      b_conv.reshape(1, 1024), W1, b1.reshape(1, 32), W5, b5.reshape(1, 16),
      W6, b6.reshape(1, 1), W7, b7.reshape(1, 1))
    return out


# speedup vs baseline: 1.2356x; 1.2356x over previous
import functools

import jax
import jax.numpy as jnp
from jax.experimental import pallas as pl
from jax.experimental.pallas import tpu as pltpu


B, N, F = 256, 256, 53
NF = 36
BB = 64
GRID = B // BB

f32 = jnp.float32
bf16 = jnp.bfloat16


def _dot(a, b, dims=(((1,), (0,)), ((), ()))):
    return jax.lax.dot_general(a, b, dims, preferred_element_type=f32)


def _fused_kernel(x_ref, phys_ref, wrt_ref, bmat_ref, wc_ref, bc_ref, w1_ref,
                  b1_ref, w5_ref, b5_ref, w6_ref, b6_ref, w7_ref, b7_ref,
                  out_ref, pool_ref, a_ref):
    step = pl.program_id(0)

    feats = x_ref[:, :, :NF].reshape(BB * N, NF).astype(bf16)
    fwt = jax.lax.dot_general(wrt_ref[...].astype(bf16), feats,
                              (((1,), (1,)), ((), ())),
                              preferred_element_type=f32)
    bmat = bmat_ref[...]
    ones_row = jnp.ones((1, N), dtype=bf16)

    LAG = 3
    pooled_cols = []
    for i in range(BB + LAG):
        if i < BB:
            f = feats[i * N:(i + 1) * N, :]
            g = jax.lax.dot_general(f, f, (((1,), (1,)), ((), ())),
                                    preferred_element_type=f32)
            a_ref[i, :, :] = jnp.maximum(g, 0.0).astype(bf16)
        if i >= LAG:
            j = i - LAG
            a = a_ref[j, :, :]
            lhs = jnp.concatenate(
                [ones_row,
                 (fwt[:, j * N:(j + 1) * N] + bmat).astype(bf16)], axis=0)
            nt_full = _dot(lhs, a)
            deg = nt_full[0:1, :] + 1e-6
            r = 1.0 / deg
            nt = jnp.maximum(nt_full[1:21, :] + 1e-6 * bmat, 0.0)
            pooled_cols.append(jnp.sum(nt * r, axis=1, keepdims=True))
    pcols = jnp.concatenate(pooled_cols, axis=1)
    pool_ref[pl.ds(step * BB, BB), :] = pcols.T

    @pl.when(step == GRID - 1)
    def _head():
        p = pool_ref[...].astype(bf16)
        c = jnp.maximum(_dot(p, wc_ref[...].astype(bf16)) + bc_ref[...], 0.0)
        x1 = jnp.maximum(_dot(c.astype(bf16), w1_ref[...].astype(bf16))
                         + b1_ref[...], 0.0)
        x5 = jnp.maximum(_dot(x1.astype(bf16), w5_ref[...].astype(bf16))
                         + b5_ref[...], 0.0)
        mv = _dot(x5.astype(bf16), w6_ref[...].astype(bf16)) + b6_ref[...]
        phys = phys_ref[...]
        col0 = (mv * w7_ref[0, 0]
                + jax.lax.dot_general(phys, w7_ref[1:, :],
                                      (((1,), (0,)), ((), ())),
                                      preferred_element_type=f32)
                + b7_ref[...])
        out_ref[...] = jnp.concatenate([col0, phys], axis=1)


@functools.partial(jax.jit, static_argnames=())
def kernel(inputs, W_rule, b_rule, W_conv, b_conv, W1, b1, W5, b5, W6, b6,
           W7, b7):
    full = lambda shape: pl.BlockSpec(shape, lambda i: (0,) * len(shape))
    phys_all = inputs[:, 0, NF + 2:F]
    b_mat = jnp.broadcast_to(b_rule.reshape(20, 1), (20, N))
    out = pl.pallas_call(
        _fused_kernel,
        grid=(GRID,),
        in_specs=[
            pl.BlockSpec((BB, N, F), lambda i: (i, 0, 0)),
            full((B, 15)),
            full((20, 36)),
            full((20, N)),
            full((20, 1024)),
            full((1, 1024)),
            full((1024, 32)),
            full((1, 32)),
            full((32, 16)),
            full((1, 16)),
            full((16, 1)),
            full((1, 1)),
            full((16, 1)),
            full((1, 1)),
        ],
        out_specs=pl.BlockSpec((B, 16), lambda i: (0, 0)),
        out_shape=jax.ShapeDtypeStruct((B, 16), f32),
        scratch_shapes=[pltpu.VMEM((B, 20), f32),
                        pltpu.VMEM((BB, N, N), bf16)],
    )(inputs, phys_all, W_rule.T, b_mat, W_conv,
      b_conv.reshape(1, 1024), W1, b1.reshape(1, 32), W5, b5.reshape(1, 16),
      W6, b6.reshape(1, 1), W7, b7.reshape(1, 1))
    return out


# speedup vs baseline: 1.3562x; 1.0976x over previous
import functools

import jax
import jax.numpy as jnp
from jax.experimental import pallas as pl
from jax.experimental.pallas import tpu as pltpu


B, N, F = 256, 256, 53
NF = 36
BB = 64
GRID = B // BB

f32 = jnp.float32
bf16 = jnp.bfloat16


def _dot(a, b, dims=(((1,), (0,)), ((), ()))):
    return jax.lax.dot_general(a, b, dims, preferred_element_type=f32)


def _fused_kernel(x_ref, phys_ref, wrt_ref, bmat_ref, wc_ref, bc_ref, w1_ref,
                  b1_ref, w5_ref, b5_ref, w6_ref, b6_ref, w7_ref, b7_ref,
                  out_ref, pool_ref, a_ref, lhs_ref):
    step = pl.program_id(0)

    bmat = bmat_ref[...]
    ones_row = jnp.ones((1, N), dtype=bf16)
    wrt = wrt_ref[...].astype(bf16)

    LAG = 3
    pooled_cols = []
    for i in range(BB + LAG):
        if i < BB:
            f = x_ref[i, :, :NF].astype(bf16)
            gw = jax.lax.dot_general(jnp.concatenate([f, wrt], axis=0), f,
                                     (((1,), (1,)), ((), ())),
                                     preferred_element_type=f32)
            a_ref[i, :, :] = jnp.maximum(gw[:N, :], 0.0).astype(bf16)
            lhs_ref[i, :, :] = jnp.concatenate(
                [ones_row, (gw[N:, :] + bmat).astype(bf16)], axis=0)
        if i >= LAG:
            j = i - LAG
            nt_full = _dot(lhs_ref[j, :, :], a_ref[j, :, :])
            deg = nt_full[0:1, :] + 1e-6
            r = 1.0 / deg
            nt = jnp.maximum(nt_full[1:21, :] + 1e-6 * bmat, 0.0)
            pooled_cols.append(jnp.sum(nt * r, axis=1, keepdims=True))
    pcols = jnp.concatenate(pooled_cols, axis=1)
    pool_ref[pl.ds(step * BB, BB), :] = pcols.T

    @pl.when(step == GRID - 1)
    def _head():
        p = pool_ref[...].astype(bf16)
        c = jnp.maximum(_dot(p, wc_ref[...].astype(bf16)) + bc_ref[...], 0.0)
        x1 = jnp.maximum(_dot(c.astype(bf16), w1_ref[...].astype(bf16))
                         + b1_ref[...], 0.0)
        x5 = jnp.maximum(_dot(x1.astype(bf16), w5_ref[...].astype(bf16))
                         + b5_ref[...], 0.0)
        mv = _dot(x5.astype(bf16), w6_ref[...].astype(bf16)) + b6_ref[...]
        phys = phys_ref[...]
        col0 = (mv * w7_ref[0, 0]
                + jax.lax.dot_general(phys, w7_ref[1:, :],
                                      (((1,), (0,)), ((), ())),
                                      preferred_element_type=f32)
                + b7_ref[...])
        out_ref[...] = jnp.concatenate([col0, phys], axis=1)


@functools.partial(jax.jit, static_argnames=())
def kernel(inputs, W_rule, b_rule, W_conv, b_conv, W1, b1, W5, b5, W6, b6,
           W7, b7):
    full = lambda shape: pl.BlockSpec(shape, lambda i: (0,) * len(shape))
    phys_all = inputs[:, 0, NF + 2:F]
    b_mat = jnp.broadcast_to(b_rule.reshape(20, 1), (20, N))
    out = pl.pallas_call(
        _fused_kernel,
        grid=(GRID,),
        in_specs=[
            pl.BlockSpec((BB, N, F), lambda i: (i, 0, 0)),
            full((B, 15)),
            full((20, 36)),
            full((20, N)),
            full((20, 1024)),
            full((1, 1024)),
            full((1024, 32)),
            full((1, 32)),
            full((32, 16)),
            full((1, 16)),
            full((16, 1)),
            full((1, 1)),
            full((16, 1)),
            full((1, 1)),
        ],
        out_specs=pl.BlockSpec((B, 16), lambda i: (0, 0)),
        out_shape=jax.ShapeDtypeStruct((B, 16), f32),
        scratch_shapes=[pltpu.VMEM((B, 20), f32),
                        pltpu.VMEM((BB, N, N), bf16),
                        pltpu.VMEM((BB, 21, N), bf16)],
    )(inputs, phys_all, W_rule.T, b_mat, W_conv,
      b_conv.reshape(1, 1024), W1, b1.reshape(1, 32), W5, b5.reshape(1, 16),
      W6, b6.reshape(1, 1), W7, b7.reshape(1, 1))
    return out


# speedup vs baseline: 1.3602x; 1.0030x over previous
import functools

import jax
import jax.numpy as jnp
from jax.experimental import pallas as pl
from jax.experimental.pallas import tpu as pltpu


B, N, F = 256, 256, 53
NF = 36
BB = 64
GRID = B // BB

f32 = jnp.float32
bf16 = jnp.bfloat16


def _dot(a, b, dims=(((1,), (0,)), ((), ()))):
    return jax.lax.dot_general(a, b, dims, preferred_element_type=f32)


def _fused_kernel(x_ref, phys_ref, wrt_ref, bmat_ref, wc_ref, bc_ref, w1_ref,
                  b1_ref, w5_ref, b5_ref, w6_ref, b6_ref, w7_ref, b7_ref,
                  out_ref, pool_ref, a_ref, lhs_ref):
    step = pl.program_id(0)

    bmat = bmat_ref[...]
    bmat_eps = bmat * 1e-6
    ones_row = jnp.ones((1, N), dtype=bf16)
    wrt = wrt_ref[...].astype(bf16)

    LAG = 3
    pooled_cols = []
    for i in range(BB + LAG):
        if i < BB:
            f = x_ref[i, :, :NF].astype(bf16)
            gw = jax.lax.dot_general(jnp.concatenate([f, wrt], axis=0), f,
                                     (((1,), (1,)), ((), ())),
                                     preferred_element_type=f32)
            a_ref[i, :, :] = jnp.maximum(gw[:N, :].astype(bf16),
                                         jnp.asarray(0.0, bf16))
            lhs_ref[i, :, :] = jnp.concatenate(
                [ones_row, (gw[N:, :] + bmat).astype(bf16)], axis=0)
        if i >= LAG:
            j = i - LAG
            nt_full = _dot(lhs_ref[j, :, :], a_ref[j, :, :])
            deg = nt_full[0:1, :] + 1e-6
            r = 1.0 / deg
            nt = jnp.maximum(nt_full[1:21, :] + bmat_eps, 0.0)
            pcol = jnp.sum(nt * r, axis=1, keepdims=True)
            pool_ref[step, :, j:j + 1] = pcol

    @pl.when(step == GRID - 1)
    def _head():
        p = (pool_ref[...].transpose(0, 2, 1)
             .reshape(B, 20).astype(bf16))
        c = jnp.maximum(_dot(p, wc_ref[...].astype(bf16)) + bc_ref[...], 0.0)
        x1 = jnp.maximum(_dot(c.astype(bf16), w1_ref[...].astype(bf16))
                         + b1_ref[...], 0.0)
        x5 = jnp.maximum(_dot(x1.astype(bf16), w5_ref[...].astype(bf16))
                         + b5_ref[...], 0.0)
        mv = _dot(x5.astype(bf16), w6_ref[...].astype(bf16)) + b6_ref[...]
        phys = phys_ref[...]
        col0 = (mv * w7_ref[0, 0]
                + jax.lax.dot_general(phys, w7_ref[1:, :],
                                      (((1,), (0,)), ((), ())),
                                      preferred_element_type=f32)
                + b7_ref[...])
        out_ref[...] = jnp.concatenate([col0, phys], axis=1)


@functools.partial(jax.jit, static_argnames=())
def kernel(inputs, W_rule, b_rule, W_conv, b_conv, W1, b1, W5, b5, W6, b6,
           W7, b7):
    full = lambda shape: pl.BlockSpec(shape, lambda i: (0,) * len(shape))
    phys_all = inputs[:, 0, NF + 2:F]
    b_mat = jnp.broadcast_to(b_rule.reshape(20, 1), (20, N))
    out = pl.pallas_call(
        _fused_kernel,
        grid=(GRID,),
        in_specs=[
            pl.BlockSpec((BB, N, F), lambda i: (i, 0, 0)),
            full((B, 15)),
            full((20, 36)),
            full((20, N)),
            full((20, 1024)),
            full((1, 1024)),
            full((1024, 32)),
            full((1, 32)),
            full((32, 16)),
            full((1, 16)),
            full((16, 1)),
            full((1, 1)),
            full((16, 1)),
            full((1, 1)),
        ],
        out_specs=pl.BlockSpec((B, 16), lambda i: (0, 0)),
        out_shape=jax.ShapeDtypeStruct((B, 16), f32),
        scratch_shapes=[pltpu.VMEM((GRID, 20, BB), f32),
                        pltpu.VMEM((BB, N, N), bf16),
                        pltpu.VMEM((BB, 21, N), bf16)],
    )(inputs, phys_all, W_rule.T, b_mat, W_conv,
      b_conv.reshape(1, 1024), W1, b1.reshape(1, 32), W5, b5.reshape(1, 16),
      W6, b6.reshape(1, 1), W7, b7.reshape(1, 1))
    return out


# speedup vs baseline: 1.6146x; 1.1870x over previous
import functools

import jax
import jax.numpy as jnp
from jax.experimental import pallas as pl
from jax.experimental.pallas import tpu as pltpu


B, N, F = 256, 256, 53
NF = 36
BB = 256
GRID = B // BB

f32 = jnp.float32
bf16 = jnp.bfloat16


def _dot(a, b, dims=(((1,), (0,)), ((), ()))):
    return jax.lax.dot_general(a, b, dims, preferred_element_type=f32)


def _fused_kernel(x_ref, phys_ref, wrt_ref, bmat_ref, wc_ref, bc_ref, w1_ref,
                  b1_ref, w5_ref, b5_ref, w6_ref, b6_ref, w7_ref, b7_ref,
                  out_ref, pool_ref, a_ref, lhs_ref):
    step = pl.program_id(0)

    bmat = bmat_ref[...]
    bmat_eps = bmat * 1e-6
    ones_row = jnp.ones((1, N), dtype=bf16)
    wrt = wrt_ref[...].astype(bf16)

    LAG = 3
    pooled_cols = []
    for i in range(BB + LAG):
        if i < BB:
            f = x_ref[i, :, :]
            gw = jax.lax.dot_general(jnp.concatenate([f, wrt], axis=0), f,
                                     (((1,), (1,)), ((), ())),
                                     preferred_element_type=f32)
            a_ref[i, :, :] = jnp.maximum(gw[:N, :].astype(bf16),
                                         jnp.asarray(0.0, bf16))
            lhs_ref[i, :, :] = jnp.concatenate(
                [ones_row, (gw[N:, :] + bmat).astype(bf16)], axis=0)
        if i >= LAG:
            j = i - LAG
            nt_full = _dot(lhs_ref[j, :, :], a_ref[j, :, :])
            deg = nt_full[0:1, :] + 1e-6
            r = 1.0 / deg
            nt = jnp.maximum(nt_full[1:21, :] + bmat_eps, 0.0)
            pcol = jnp.sum(nt * r, axis=1, keepdims=True)
            pool_ref[step, :, j:j + 1] = pcol

    @pl.when(step == GRID - 1)
    def _head():
        p = (pool_ref[...].transpose(0, 2, 1)
             .reshape(B, 20).astype(bf16))
        c = jnp.maximum(_dot(p, wc_ref[...].astype(bf16)) + bc_ref[...], 0.0)
        x1 = jnp.maximum(_dot(c.astype(bf16), w1_ref[...].astype(bf16))
                         + b1_ref[...], 0.0)
        x5 = jnp.maximum(_dot(x1.astype(bf16), w5_ref[...].astype(bf16))
                         + b5_ref[...], 0.0)
        mv = _dot(x5.astype(bf16), w6_ref[...].astype(bf16)) + b6_ref[...]
        phys = phys_ref[...]
        col0 = (mv * w7_ref[0, 0]
                + jax.lax.dot_general(phys, w7_ref[1:, :],
                                      (((1,), (0,)), ((), ())),
                                      preferred_element_type=f32)
                + b7_ref[...])
        out_ref[...] = jnp.concatenate([col0, phys], axis=1)


@functools.partial(jax.jit, static_argnames=())
def kernel(inputs, W_rule, b_rule, W_conv, b_conv, W1, b1, W5, b5, W6, b6,
           W7, b7):
    full = lambda shape: pl.BlockSpec(shape, lambda i: (0,) * len(shape))
    phys_all = inputs[:, 0, NF + 2:F]
    b_mat = jnp.broadcast_to(b_rule.reshape(20, 1), (20, N))
    out = pl.pallas_call(
        _fused_kernel,
        grid=(GRID,),
        in_specs=[
            pl.BlockSpec((BB, N, NF), lambda i: (i, 0, 0)),
            full((B, 15)),
            full((20, 36)),
            full((20, N)),
            full((20, 1024)),
            full((1, 1024)),
            full((1024, 32)),
            full((1, 32)),
            full((32, 16)),
            full((1, 16)),
            full((16, 1)),
            full((1, 1)),
            full((16, 1)),
            full((1, 1)),
        ],
        out_specs=pl.BlockSpec((B, 16), lambda i: (0, 0)),
        out_shape=jax.ShapeDtypeStruct((B, 16), f32),
        scratch_shapes=[pltpu.VMEM((GRID, 20, BB), f32),
                        pltpu.VMEM((BB, N, N), bf16),
                        pltpu.VMEM((BB, 21, N), bf16)],
    )(inputs[:, :, :NF].astype(bf16), phys_all, W_rule.T, b_mat, W_conv,
      b_conv.reshape(1, 1024), W1, b1.reshape(1, 32), W5, b5.reshape(1, 16),
      W6, b6.reshape(1, 1), W7, b7.reshape(1, 1))
    return out


# speedup vs baseline: 1.7371x; 1.0759x over previous
import functools

import jax
import jax.numpy as jnp
from jax.experimental import pallas as pl
from jax.experimental.pallas import tpu as pltpu


B, N, F = 256, 256, 53
NF = 36
BB = 64
GRID = B // BB

f32 = jnp.float32
bf16 = jnp.bfloat16


def _dot(a, b, dims=(((1,), (0,)), ((), ()))):
    return jax.lax.dot_general(a, b, dims, preferred_element_type=f32)


def _fused_kernel(x_ref, phys_ref, wrt_ref, bmat_ref, wc_ref, bc_ref, w1_ref,
                  b1_ref, w5_ref, b5_ref, w6_ref, b6_ref, w7_ref, b7_ref,
                  out_ref, pool_ref, a_ref, lhs_ref):
    step = pl.program_id(0)

    bmat = bmat_ref[...]
    bmat_eps = bmat * 1e-6
    ones_row = jnp.ones((1, N), dtype=bf16)
    wrt = wrt_ref[...].astype(bf16)

    LAG = 3
    pooled_cols = []
    for i in range(BB + LAG):
        if i < BB:
            f = x_ref[i, :, :]
            gw = jax.lax.dot_general(jnp.concatenate([f, wrt], axis=0), f,
                                     (((1,), (1,)), ((), ())),
                                     preferred_element_type=f32)
            a_ref[i, :, :] = jnp.maximum(gw[:N, :].astype(bf16),
                                         jnp.asarray(0.0, bf16))
            lhs_ref[i, :, :] = jnp.concatenate(
                [ones_row, (gw[N:, :] + bmat).astype(bf16)], axis=0)
        if i >= LAG:
            j = i - LAG
            nt_full = _dot(lhs_ref[j, :, :], a_ref[j, :, :])
            deg = nt_full[0:1, :] + 1e-6
            r = 1.0 / deg
            nt = jnp.maximum(nt_full[1:21, :] + bmat_eps, 0.0)
            pcol = jnp.sum(nt * r, axis=1, keepdims=True)
            pool_ref[step, :, j:j + 1] = pcol

    @pl.when(step == GRID - 1)
    def _head():
        p = (pool_ref[...].transpose(0, 2, 1)
             .reshape(B, 20).astype(bf16))
        c = jnp.maximum(_dot(p, wc_ref[...].astype(bf16)) + bc_ref[...], 0.0)
        x1 = jnp.maximum(_dot(c.astype(bf16), w1_ref[...].astype(bf16))
                         + b1_ref[...], 0.0)
        x5 = jnp.maximum(_dot(x1.astype(bf16), w5_ref[...].astype(bf16))
                         + b5_ref[...], 0.0)
        mv = _dot(x5.astype(bf16), w6_ref[...].astype(bf16)) + b6_ref[...]
        phys = phys_ref[...]
        col0 = (mv * w7_ref[0, 0]
                + jax.lax.dot_general(phys, w7_ref[1:, :],
                                      (((1,), (0,)), ((), ())),
                                      preferred_element_type=f32)
                + b7_ref[...])
        out_ref[...] = jnp.concatenate([col0, phys], axis=1)


@functools.partial(jax.jit, static_argnames=())
def kernel(inputs, W_rule, b_rule, W_conv, b_conv, W1, b1, W5, b5, W6, b6,
           W7, b7):
    full = lambda shape: pl.BlockSpec(shape, lambda i: (0,) * len(shape))
    phys_all = inputs[:, 0, NF + 2:F]
    b_mat = jnp.broadcast_to(b_rule.reshape(20, 1), (20, N))
    out = pl.pallas_call(
        _fused_kernel,
        grid=(GRID,),
        in_specs=[
            pl.BlockSpec((BB, N, NF), lambda i: (i, 0, 0)),
            full((B, 15)),
            full((20, 36)),
            full((20, N)),
            full((20, 1024)),
            full((1, 1024)),
            full((1024, 32)),
            full((1, 32)),
            full((32, 16)),
            full((1, 16)),
            full((16, 1)),
            full((1, 1)),
            full((16, 1)),
            full((1, 1)),
        ],
        out_specs=pl.BlockSpec((B, 16), lambda i: (0, 0)),
        out_shape=jax.ShapeDtypeStruct((B, 16), f32),
        scratch_shapes=[pltpu.VMEM((GRID, 20, BB), f32),
                        pltpu.VMEM((BB, N, N), bf16),
                        pltpu.VMEM((BB, 21, N), bf16)],
    )(inputs[:, :, :NF].astype(bf16), phys_all, W_rule.T, b_mat, W_conv,
      b_conv.reshape(1, 1024), W1, b1.reshape(1, 32), W5, b5.reshape(1, 16),
      W6, b6.reshape(1, 1), W7, b7.reshape(1, 1))
    return out
